# baseline (speedup 1.0000x reference)
import dataclasses
import functools

import jax
import jax.numpy as jnp
from jax import lax
from jax.experimental import pallas as pl
from jax.experimental.pallas import tpu as pltpu
from jax.experimental.pallas import tpu_sc as plsc

NC = 2
NS = 16
NW = NC * NS
CHUNK = 128
ACH = 64
IB = 16

N_ACC = 10240
HR = N_ACC // 128

_vector_mesh = plsc.VectorSubcoreMesh(core_axis_name="c", subcore_axis_name="s")

_scatter_cp = pltpu.CompilerParams()
if "needs_layout_passes" in pltpu.CompilerParams.__dataclass_fields__:
    _scatter_cp = dataclasses.replace(_scatter_cp, needs_layout_passes=False)


def _sc_hist(nch):
    @functools.partial(
        pl.kernel,
        out_type=jax.ShapeDtypeStruct((NW, HR, 128), jnp.float32),
        mesh=_vector_mesh,
        compiler_params=_scatter_cp,
        scratch_types=[
            pltpu.VMEM((nch, CHUNK), jnp.int32),
            pltpu.VMEM((HR, 128), jnp.float32),
        ],
    )
    def hist(col3_hbm, out_hbm, idx_v, hist_v):
        cid = lax.axis_index("c")
        sid = lax.axis_index("s")
        wid = cid * NS + sid

        pltpu.sync_copy(col3_hbm.at[wid], idx_v)

        @pl.loop(0, HR)
        def _(r):
            @pl.loop(0, 8)
            def _(i):
                hist_v[r, pl.ds(i * 16, 16)] = jnp.zeros((16,), jnp.float32)

        ones16 = jnp.ones((16,), jnp.float32)

        @pl.loop(0, nch)
        def _(j):
            @pl.loop(0, CHUNK // 16)
            def _(i):
                iv = idx_v[j, pl.ds(i * 16, 16)]
                r = lax.shift_right_logical(iv, 7)
                c = lax.bitwise_and(iv, 127)
                plsc.addupdate_scatter(hist_v, [r, c], ones16)

        pltpu.sync_copy(hist_v, out_hbm.at[wid])

    return hist


def _sc_agg(nchunks, d):
    assert nchunks % IB == 0 and nchunks % 4 == 0

    @functools.partial(
        pl.kernel,
        out_type=jax.ShapeDtypeStruct((N_ACC, d), jnp.float32),
        mesh=_vector_mesh,
        scratch_types=[
            pltpu.VMEM((2, IB, ACH), jnp.int32),
            pltpu.VMEM((2, IB, ACH), jnp.int32),
            [pltpu.VMEM((ACH, d), jnp.float32) for _ in range(4)],
            pltpu.VMEM_SHARED((N_ACC, d), jnp.float32),
            [pltpu.SemaphoreType.DMA for _ in range(4)],
            [pltpu.SemaphoreType.DMA for _ in range(4)],
            pltpu.SemaphoreType.DMA,
        ],
    )
    def agg(row3_hbm, col3_hbm, zeros_hbm, g_hbm, out_hbm,
            row_v, col_v, bufs, acc, gsems, ssems, isem):
        cid = lax.axis_index("c")
        sid = lax.axis_index("s")
        rows_per_tile = N_ACC // NS
        base = sid * rows_per_tile

        def pipeline():
            nblk = nchunks // IB

            def issue_idx(blk, slot):
                pltpu.async_copy(row3_hbm.at[sid, pl.ds(blk * IB, IB)],
                                 row_v.at[slot], isem)
                pltpu.async_copy(col3_hbm.at[sid, pl.ds(blk * IB, IB)],
                                 col_v.at[slot], isem)

            def drain_idx():
                pltpu.make_async_copy(row3_hbm.at[sid, pl.ds(0, IB)],
                                      row_v.at[0], isem).wait()
                pltpu.make_async_copy(col3_hbm.at[sid, pl.ds(0, IB)],
                                      col_v.at[0], isem).wait()

            def gather(c, buf, sem):
                slot = lax.bitwise_and(lax.shift_right_logical(c, 4), 1)
                pltpu.async_copy(
                    g_hbm.at[row_v.at[slot, lax.bitwise_and(c, IB - 1)]],
                    buf, sem)

            def wait_gather(buf, sem):
                pltpu.make_async_copy(g_hbm.at[row_v.at[0, 0]], buf, sem).wait()

            def wait_scatter(buf, sem):
                pltpu.make_async_copy(buf, acc.at[col_v.at[0, 0]], sem).wait()

            issue_idx(0, 0)
            drain_idx()

            gather(0, bufs[0], gsems[0])
            gather(1, bufs[1], gsems[1])

            @pl.loop(0, nchunks // 4)
            def _(k4):
                k_base = k4 * 4
                for b in range(4):
                    k = k_base + b
                    if b == 0:
                        @pl.when(lax.bitwise_and(k4, 3) == 0)
                        def _():
                            blk = lax.shift_right_logical(k4, 2)
                            @pl.when(blk + 1 < nblk)
                            def _():
                                issue_idx(blk + 1, lax.bitwise_and(blk + 1, 1))
                    kk = k + 2
                    bb = (b + 2) % 4

                    @pl.when(kk < nchunks)
                    def _():
                        @pl.when(lax.bitwise_and(kk, IB - 1) == 0)
                        def _():
                            drain_idx()
                        @pl.when(k >= 2)
                        def _():
                            wait_scatter(bufs[bb], ssems[bb])
                        gather(kk, bufs[bb], gsems[bb])

                    slot_k = lax.bitwise_and(lax.shift_right_logical(k, 4), 1)
                    jj = lax.bitwise_and(k, IB - 1)
                    wait_gather(bufs[b], gsems[b])
                    pltpu.async_copy(bufs[b], acc.at[col_v.at[slot_k, jj]],
                                     ssems[b], add=True)

            for b in range(4):
                wait_scatter(bufs[b], ssems[b])

        @pl.when(cid == 0)
        def _():
            with jax.named_scope("acc_zero"):
                pltpu.sync_copy(zeros_hbm.at[pl.ds(base, rows_per_tile)],
                                acc.at[pl.ds(base, rows_per_tile)])
                plsc.subcore_barrier()
            with jax.named_scope("edge_pipe"):
                pipeline()
                plsc.subcore_barrier()
            with jax.named_scope("acc_writeout"):
                pltpu.sync_copy(acc.at[pl.ds(base, rows_per_tile)],
                                out_hbm.at[pl.ds(base, rows_per_tile)])

    return agg


def _tc_matmul(x_ref, w_ref, o_ref):
    o_ref[...] = jnp.dot(x_ref[...], w_ref[...],
                         preferred_element_type=jnp.float32,
                         precision=lax.Precision.HIGHEST)


def _tc_deg(degp_ref, o_ref):
    o_ref[...] = lax.rsqrt(jnp.sum(degp_ref[...], axis=0) + 1.0)


def _tc_scale(h_ref, dinv_ref, o_ref):
    o_ref[...] = h_ref[...] * dinv_ref[...]


def _tc_layer2(acc_ref, g_ref, dinv_ref, b_ref, w_ref, o_ref):
    n = g_ref.shape[0]
    s = acc_ref[:n, :] + g_ref[...]
    z = jax.nn.relu(s * dinv_ref[...] + b_ref[...])
    o_ref[...] = jnp.dot(z, w_ref[...],
                         preferred_element_type=jnp.float32,
                         precision=lax.Precision.HIGHEST) * dinv_ref[...]


def _tc_final(acc_ref, g_ref, dinv_ref, b_ref, o_ref):
    n = g_ref.shape[0]
    s = acc_ref[:n, :] + g_ref[...]
    o_ref[...] = jax.nn.relu(s * dinv_ref[...] + b_ref[...])


def _tc_call(body, out_shape, *args):
    return pl.pallas_call(
        body,
        out_shape=jax.ShapeDtypeStruct(out_shape, jnp.float32),
    )(*args)


@jax.jit
def kernel(x, edge_index, W1, b1, W2, b2):
    n, d_in = x.shape
    d_hid = W1.shape[1]
    d_out = W2.shape[1]
    e = edge_index.shape[1]

    per = NW * ACH * IB
    e_pad = ((e + per - 1) // per) * per
    nch_h = e_pad // (NW * CHUNK)
    row_flat = jnp.concatenate([edge_index[0], jnp.zeros((e_pad - e,), jnp.int32)])
    col_flat = jnp.concatenate([edge_index[1], jnp.full((e_pad - e,), n, jnp.int32)])
    col_h = col_flat.reshape(NW, nch_h, CHUNK)

    tot = e_pad // (NS * ACH)
    row0 = row_flat.reshape(NS, tot, ACH)
    col0 = col_flat.reshape(NS, tot, ACH)

    zeros_acc = jnp.zeros((N_ACC, d_hid), jnp.float32)

    b1r = b1.reshape(1, d_hid)
    b2r = b2.reshape(1, d_out)

    degp = _sc_hist(nch_h)(col_h)
    h1 = _tc_call(_tc_matmul, (n, d_hid), x, W1)
    dinv2d = _tc_call(_tc_deg, (HR, 128), degp)
    dinv = dinv2d.reshape(N_ACC, 1)[:n]
    g1 = _tc_call(_tc_scale, (n, d_hid), h1, dinv)
    acc1 = _sc_agg(tot, d_hid)(row0, col0, zeros_acc, g1)
    g2 = _tc_call(_tc_layer2, (n, d_out), acc1, g1, dinv, b1r, W2)
    acc2 = _sc_agg(tot, d_out)(row0, col0, zeros_acc, g2)
    out = _tc_call(_tc_final, (n, d_out), acc2, g2, dinv, b2r)
    return out


# speedup vs baseline: 1.1708x; 1.1708x over previous
import dataclasses
import functools

import jax
import jax.numpy as jnp
from jax import lax
from jax.experimental import pallas as pl
from jax.experimental.pallas import tpu as pltpu
from jax.experimental.pallas import tpu_sc as plsc

NC = 2
NS = 16
NW = NC * NS
CHUNK = 128
ACH = 64
IB = 16

N_ACC = 10240
HR = N_ACC // 128

_vector_mesh = plsc.VectorSubcoreMesh(core_axis_name="c", subcore_axis_name="s")

_scatter_cp = pltpu.CompilerParams()
if "needs_layout_passes" in pltpu.CompilerParams.__dataclass_fields__:
    _scatter_cp = dataclasses.replace(_scatter_cp, needs_layout_passes=False)


def _sc_hist(nch):
    @functools.partial(
        pl.kernel,
        out_type=jax.ShapeDtypeStruct((NW, HR, 128), jnp.float32),
        mesh=_vector_mesh,
        compiler_params=_scatter_cp,
        scratch_types=[
            pltpu.VMEM((nch, CHUNK), jnp.int32),
            pltpu.VMEM((HR, 128), jnp.float32),
        ],
    )
    def hist(col3_hbm, out_hbm, idx_v, hist_v):
        cid = lax.axis_index("c")
        sid = lax.axis_index("s")
        wid = cid * NS + sid

        pltpu.sync_copy(col3_hbm.at[wid], idx_v)

        @pl.loop(0, HR)
        def _(r):
            @pl.loop(0, 8)
            def _(i):
                hist_v[r, pl.ds(i * 16, 16)] = jnp.zeros((16,), jnp.float32)

        ones16 = jnp.ones((16,), jnp.float32)

        @pl.loop(0, nch)
        def _(j):
            @pl.loop(0, CHUNK // 16)
            def _(i):
                iv = idx_v[j, pl.ds(i * 16, 16)]
                r = lax.shift_right_logical(iv, 7)
                c = lax.bitwise_and(iv, 127)
                plsc.addupdate_scatter(hist_v, [r, c], ones16)

        pltpu.sync_copy(hist_v, out_hbm.at[wid])

    return hist


def _sc_agg(n_c0, n_c1, d):
    for nc in (n_c0, n_c1):
        assert nc % IB == 0 and nc % 4 == 0

    @functools.partial(
        pl.kernel,
        out_type=jax.ShapeDtypeStruct((NC, N_ACC, d), jnp.float32),
        mesh=_vector_mesh,
        scratch_types=[
            pltpu.VMEM((2, IB, ACH), jnp.int32),
            pltpu.VMEM((2, IB, ACH), jnp.int32),
            [pltpu.VMEM((ACH, d), jnp.float32) for _ in range(4)],
            pltpu.VMEM_SHARED((N_ACC, d), jnp.float32),
            [pltpu.SemaphoreType.DMA for _ in range(4)],
            [pltpu.SemaphoreType.DMA for _ in range(4)],
            pltpu.SemaphoreType.DMA,
        ],
    )
    def agg(row0_hbm, col0_hbm, row1_hbm, col1_hbm, zeros_hbm, g_hbm, out_hbm,
            row_v, col_v, bufs, acc, gsems, ssems, isem):
        cid = lax.axis_index("c")
        sid = lax.axis_index("s")
        rows_per_tile = N_ACC // NS
        base = sid * rows_per_tile

        with jax.named_scope("acc_zero"):
            pltpu.sync_copy(zeros_hbm.at[pl.ds(base, rows_per_tile)],
                            acc.at[pl.ds(base, rows_per_tile)])
            plsc.subcore_barrier()

        def pipeline(nchunks, row3_hbm, col3_hbm):
            nblk = nchunks // IB

            def issue_idx(blk, slot):
                pltpu.async_copy(row3_hbm.at[sid, pl.ds(blk * IB, IB)],
                                 row_v.at[slot], isem)
                pltpu.async_copy(col3_hbm.at[sid, pl.ds(blk * IB, IB)],
                                 col_v.at[slot], isem)

            def drain_idx():
                pltpu.make_async_copy(row3_hbm.at[sid, pl.ds(0, IB)],
                                      row_v.at[0], isem).wait()
                pltpu.make_async_copy(col3_hbm.at[sid, pl.ds(0, IB)],
                                      col_v.at[0], isem).wait()

            def gather(c, buf, sem):
                slot = lax.bitwise_and(lax.shift_right_logical(c, 4), 1)
                pltpu.async_copy(
                    g_hbm.at[row_v.at[slot, lax.bitwise_and(c, IB - 1)]],
                    buf, sem)

            def wait_gather(buf, sem):
                pltpu.make_async_copy(g_hbm.at[row_v.at[0, 0]], buf, sem).wait()

            def wait_scatter(buf, sem):
                pltpu.make_async_copy(buf, acc.at[col_v.at[0, 0]], sem).wait()

            issue_idx(0, 0)
            drain_idx()

            gather(0, bufs[0], gsems[0])
            gather(1, bufs[1], gsems[1])

            @pl.loop(0, nchunks // 4)
            def _(k4):
                k_base = k4 * 4
                for b in range(4):
                    k = k_base + b
                    if b == 0:
                        @pl.when(lax.bitwise_and(k4, 3) == 0)
                        def _():
                            blk = lax.shift_right_logical(k4, 2)
                            @pl.when(blk + 1 < nblk)
                            def _():
                                issue_idx(blk + 1, lax.bitwise_and(blk + 1, 1))
                    kk = k + 2
                    bb = (b + 2) % 4

                    @pl.when(kk < nchunks)
                    def _():
                        @pl.when(lax.bitwise_and(kk, IB - 1) == 0)
                        def _():
                            drain_idx()
                        @pl.when(k >= 2)
                        def _():
                            wait_scatter(bufs[bb], ssems[bb])
                        gather(kk, bufs[bb], gsems[bb])

                    slot_k = lax.bitwise_and(lax.shift_right_logical(k, 4), 1)
                    jj = lax.bitwise_and(k, IB - 1)
                    wait_gather(bufs[b], gsems[b])
                    pltpu.async_copy(bufs[b], acc.at[col_v.at[slot_k, jj]],
                                     ssems[b], add=True)

            for b in range(4):
                wait_scatter(bufs[b], ssems[b])

        with jax.named_scope("edge_pipe"):
            @pl.when(cid == 0)
            def _():
                pipeline(n_c0, row0_hbm, col0_hbm)

            @pl.when(cid == 1)
            def _():
                pipeline(n_c1, row1_hbm, col1_hbm)

            plsc.subcore_barrier()

        with jax.named_scope("acc_writeout"):
            pltpu.sync_copy(acc.at[pl.ds(base, rows_per_tile)],
                            out_hbm.at[cid, pl.ds(base, rows_per_tile)])

    return agg


def _tc_matmul(x_ref, w_ref, o_ref):
    o_ref[...] = jnp.dot(x_ref[...], w_ref[...],
                         preferred_element_type=jnp.float32,
                         precision=lax.Precision.HIGHEST)


def _tc_deg(degp_ref, o_ref):
    o_ref[...] = lax.rsqrt(jnp.sum(degp_ref[...], axis=0) + 1.0)


def _tc_scale(h_ref, dinv_ref, o_ref):
    o_ref[...] = h_ref[...] * dinv_ref[...]


def _tc_layer2(acc_ref, g_ref, dinv_ref, b_ref, w_ref, o_ref):
    n = g_ref.shape[0]
    s = acc_ref[0, :n, :] + acc_ref[1, :n, :] + g_ref[...]
    z = jax.nn.relu(s * dinv_ref[...] + b_ref[...])
    o_ref[...] = jnp.dot(z, w_ref[...],
                         preferred_element_type=jnp.float32,
                         precision=lax.Precision.HIGHEST) * dinv_ref[...]


def _tc_final(acc_ref, g_ref, dinv_ref, b_ref, o_ref):
    n = g_ref.shape[0]
    s = acc_ref[0, :n, :] + acc_ref[1, :n, :] + g_ref[...]
    o_ref[...] = jax.nn.relu(s * dinv_ref[...] + b_ref[...])


def _tc_call(body, out_shape, *args):
    return pl.pallas_call(
        body,
        out_shape=jax.ShapeDtypeStruct(out_shape, jnp.float32),
    )(*args)


@jax.jit
def kernel(x, edge_index, W1, b1, W2, b2):
    n, d_in = x.shape
    d_hid = W1.shape[1]
    d_out = W2.shape[1]
    e = edge_index.shape[1]

    per = NW * ACH * IB
    e_pad = ((e + per - 1) // per) * per
    nch_h = e_pad // (NW * CHUNK)
    row_flat = jnp.concatenate([edge_index[0], jnp.zeros((e_pad - e,), jnp.int32)])
    col_flat = jnp.concatenate([edge_index[1], jnp.full((e_pad - e,), n, jnp.int32)])
    col_h = col_flat.reshape(NW, nch_h, CHUNK)

    tot = e_pad // (NS * ACH)
    n_c0 = (tot // 2 // IB) * IB
    n_c1 = tot - n_c0
    cut = NS * n_c0 * ACH
    row0 = row_flat[:cut].reshape(NS, n_c0, ACH)
    col0 = col_flat[:cut].reshape(NS, n_c0, ACH)
    row1 = row_flat[cut:].reshape(NS, n_c1, ACH)
    col1 = col_flat[cut:].reshape(NS, n_c1, ACH)

    zeros_acc = jnp.zeros((N_ACC, d_hid), jnp.float32)

    b1r = b1.reshape(1, d_hid)
    b2r = b2.reshape(1, d_out)

    degp = _sc_hist(nch_h)(col_h)
    h1 = _tc_call(_tc_matmul, (n, d_hid), x, W1)
    dinv2d = _tc_call(_tc_deg, (HR, 128), degp)
    dinv = dinv2d.reshape(N_ACC, 1)[:n]
    g1 = _tc_call(_tc_scale, (n, d_hid), h1, dinv)
    acc1 = _sc_agg(n_c0, n_c1, d_hid)(row0, col0, row1, col1, zeros_acc, g1)
    g2 = _tc_call(_tc_layer2, (n, d_out), acc1, g1, dinv, b1r, W2)
    acc2 = _sc_agg(n_c0, n_c1, d_out)(row0, col0, row1, col1, zeros_acc, g2)
    out = _tc_call(_tc_final, (n, d_out), acc2, g2, dinv, b2r)
    return out


# speedup vs baseline: 1.2501x; 1.0677x over previous
import dataclasses
import functools

import jax
import jax.numpy as jnp
from jax import lax
from jax.experimental import pallas as pl
from jax.experimental.pallas import tpu as pltpu
from jax.experimental.pallas import tpu_sc as plsc

NC = 2
NS = 16
NW = NC * NS
CHUNK = 128
ACH = 64
IB = 16

N_ACC = 10240
HR = N_ACC // 128

_vector_mesh = plsc.VectorSubcoreMesh(core_axis_name="c", subcore_axis_name="s")

_scatter_cp = pltpu.CompilerParams()
if "needs_layout_passes" in pltpu.CompilerParams.__dataclass_fields__:
    _scatter_cp = dataclasses.replace(_scatter_cp, needs_layout_passes=False)


def _sc_hist(nch):
    @functools.partial(
        pl.kernel,
        out_type=jax.ShapeDtypeStruct((NW, HR, 128), jnp.float32),
        mesh=_vector_mesh,
        compiler_params=_scatter_cp,
        scratch_types=[
            pltpu.VMEM((nch, CHUNK), jnp.int32),
            pltpu.VMEM((HR, 128), jnp.float32),
        ],
    )
    def hist(col3_hbm, out_hbm, idx_v, hist_v):
        cid = lax.axis_index("c")
        sid = lax.axis_index("s")
        wid = cid * NS + sid

        pltpu.sync_copy(col3_hbm.at[wid], idx_v)

        @pl.loop(0, HR)
        def _(r):
            @pl.loop(0, 8)
            def _(i):
                hist_v[r, pl.ds(i * 16, 16)] = jnp.zeros((16,), jnp.float32)

        ones16 = jnp.ones((16,), jnp.float32)

        @pl.loop(0, nch)
        def _(j):
            @pl.loop(0, CHUNK // 16)
            def _(i):
                iv = idx_v[j, pl.ds(i * 16, 16)]
                r = lax.shift_right_logical(iv, 7)
                c = lax.bitwise_and(iv, 127)
                plsc.addupdate_scatter(hist_v, [r, c], ones16)

        pltpu.sync_copy(hist_v, out_hbm.at[wid])

    return hist


def _sc_agg(n_chunks, d):
    assert n_chunks % IB == 0 and n_chunks % 4 == 0

    @functools.partial(
        pl.kernel,
        out_type=jax.ShapeDtypeStruct((NC, N_ACC, d), jnp.float32),
        mesh=_vector_mesh,
        scratch_types=[
            pltpu.VMEM((2, IB, ACH), jnp.int32),
            pltpu.VMEM((2, IB, ACH), jnp.int32),
            [pltpu.VMEM((ACH, d), jnp.float32) for _ in range(4)],
            pltpu.VMEM_SHARED((N_ACC, d), jnp.float32),
            [pltpu.SemaphoreType.DMA for _ in range(4)],
            [pltpu.SemaphoreType.DMA for _ in range(4)],
            pltpu.SemaphoreType.DMA,
        ],
    )
    def agg(row3_hbm, col3_hbm, zeros_hbm, g_hbm, out_hbm,
            row_v, col_v, bufs, acc, gsems, ssems, isem):
        cid = lax.axis_index("c")
        sid = lax.axis_index("s")
        wid = cid * NS + sid
        rows_per_tile = N_ACC // NS
        base = sid * rows_per_tile

        pltpu.sync_copy(zeros_hbm.at[pl.ds(base, rows_per_tile)],
                        acc.at[pl.ds(base, rows_per_tile)])
        plsc.subcore_barrier()

        def pipeline(nchunks):
            nblk = nchunks // IB

            def issue_idx(blk, slot):
                pltpu.async_copy(row3_hbm.at[wid, pl.ds(blk * IB, IB)],
                                 row_v.at[slot], isem)
                pltpu.async_copy(col3_hbm.at[wid, pl.ds(blk * IB, IB)],
                                 col_v.at[slot], isem)

            def drain_idx():
                pltpu.make_async_copy(row3_hbm.at[wid, pl.ds(0, IB)],
                                      row_v.at[0], isem).wait()
                pltpu.make_async_copy(col3_hbm.at[wid, pl.ds(0, IB)],
                                      col_v.at[0], isem).wait()

            def gather(c, buf, sem):
                slot = lax.bitwise_and(lax.shift_right_logical(c, 4), 1)
                pltpu.async_copy(
                    g_hbm.at[row_v.at[slot, lax.bitwise_and(c, IB - 1)]],
                    buf, sem)

            def wait_gather(buf, sem):
                pltpu.make_async_copy(g_hbm.at[row_v.at[0, 0]], buf, sem).wait()

            def wait_scatter(buf, sem):
                pltpu.make_async_copy(buf, acc.at[col_v.at[0, 0]], sem).wait()

            issue_idx(0, 0)
            drain_idx()

            gather(0, bufs[0], gsems[0])
            gather(1, bufs[1], gsems[1])

            @pl.loop(0, nchunks // 4)
            def _(k4):
                k_base = k4 * 4
                for b in range(4):
                    k = k_base + b
                    if b == 0:
                        @pl.when(lax.bitwise_and(k4, 3) == 0)
                        def _():
                            blk = lax.shift_right_logical(k4, 2)
                            @pl.when(blk + 1 < nblk)
                            def _():
                                issue_idx(blk + 1, lax.bitwise_and(blk + 1, 1))
                    kk = k + 2
                    bb = (b + 2) % 4

                    @pl.when(kk < nchunks)
                    def _():
                        @pl.when(lax.bitwise_and(kk, IB - 1) == 0)
                        def _():
                            drain_idx()
                        @pl.when(k >= 2)
                        def _():
                            wait_scatter(bufs[bb], ssems[bb])
                        gather(kk, bufs[bb], gsems[bb])

                    slot_k = lax.bitwise_and(lax.shift_right_logical(k, 4), 1)
                    jj = lax.bitwise_and(k, IB - 1)
                    wait_gather(bufs[b], gsems[b])
                    pltpu.async_copy(bufs[b], acc.at[col_v.at[slot_k, jj]],
                                     ssems[b], add=True)

            for b in range(4):
                wait_scatter(bufs[b], ssems[b])

        pipeline(n_chunks)
        plsc.subcore_barrier()
        pltpu.sync_copy(acc.at[pl.ds(base, rows_per_tile)],
                        out_hbm.at[cid, pl.ds(base, rows_per_tile)])

    return agg


def _tc_matmul(x_ref, w_ref, o_ref):
    o_ref[...] = jnp.dot(x_ref[...], w_ref[...],
                         preferred_element_type=jnp.float32,
                         precision=lax.Precision.HIGHEST)


def _tc_deg(degp_ref, o_ref):
    o_ref[...] = lax.rsqrt(jnp.sum(degp_ref[...], axis=0) + 1.0)


def _tc_scale(h_ref, dinv_ref, o_ref):
    o_ref[...] = h_ref[...] * dinv_ref[...]


def _tc_layer2(acc_ref, g_ref, dinv_ref, b_ref, w_ref, o_ref):
    n = g_ref.shape[0]
    s = acc_ref[0, :n, :] + acc_ref[1, :n, :] + g_ref[...]
    z = jax.nn.relu(s * dinv_ref[...] + b_ref[...])
    o_ref[...] = jnp.dot(z, w_ref[...],
                         preferred_element_type=jnp.float32,
                         precision=lax.Precision.HIGHEST) * dinv_ref[...]


def _tc_final(acc_ref, g_ref, dinv_ref, b_ref, o_ref):
    n = g_ref.shape[0]
    s = acc_ref[0, :n, :] + acc_ref[1, :n, :] + g_ref[...]
    o_ref[...] = jax.nn.relu(s * dinv_ref[...] + b_ref[...])


def _tc_call(body, out_shape, *args):
    return pl.pallas_call(
        body,
        out_shape=jax.ShapeDtypeStruct(out_shape, jnp.float32),
    )(*args)


@jax.jit
def kernel(x, edge_index, W1, b1, W2, b2):
    n, d_in = x.shape
    d_hid = W1.shape[1]
    d_out = W2.shape[1]
    e = edge_index.shape[1]

    per = NW * ACH * IB
    e_pad = ((e + per - 1) // per) * per
    nch_h = e_pad // (NW * CHUNK)
    row_flat = jnp.concatenate([edge_index[0], jnp.zeros((e_pad - e,), jnp.int32)])
    col_flat = jnp.concatenate([edge_index[1], jnp.full((e_pad - e,), n, jnp.int32)])
    col_h = col_flat.reshape(NW, nch_h, CHUNK)

    nchunks = e_pad // (NW * ACH)
    row3 = row_flat.reshape(NW, nchunks, ACH)
    col3 = col_flat.reshape(NW, nchunks, ACH)

    zeros_acc = jnp.zeros((N_ACC, d_hid), jnp.float32)

    b1r = b1.reshape(1, d_hid)
    b2r = b2.reshape(1, d_out)

    degp = _sc_hist(nch_h)(col_h)
    h1 = _tc_call(_tc_matmul, (n, d_hid), x, W1)
    dinv2d = _tc_call(_tc_deg, (HR, 128), degp)
    dinv = dinv2d.reshape(N_ACC, 1)[:n]
    g1 = _tc_call(_tc_scale, (n, d_hid), h1, dinv)
    acc1 = _sc_agg(nchunks, d_hid)(row3, col3, zeros_acc, g1)
    g2 = _tc_call(_tc_layer2, (n, d_out), acc1, g1, dinv, b1r, W2)
    acc2 = _sc_agg(nchunks, d_out)(row3, col3, zeros_acc, g2)
    out = _tc_call(_tc_final, (n, d_out), acc2, g2, dinv, b2r)
    return out
